# baseline (device time: 78454 ns/iter reference)
import jax
import jax.numpy as jnp
from jax import lax
from jax.experimental import pallas as pl
from jax.experimental.pallas import tpu as pltpu

N_DEV = 32


def kernel(x, router_W, route_idx, expert_W):
    T, D = x.shape
    E = router_W.shape[1]
    E_local, _, H = expert_W.shape
    rows_per = T // N_DEV

    def body(x_ref, rw_ref, idx_ref, ew_ref, out_ref,
             partial_ref, comm_ref, send_sems, recv_sems):
        my = lax.axis_index("i")
        left = lax.rem(my - 1 + N_DEV, N_DEV)
        right = lax.rem(my + 1, N_DEV)

        barrier_sem = pltpu.get_barrier_semaphore()
        for nbr in [left, right]:
            pl.semaphore_signal(
                barrier_sem, inc=1,
                device_id=(nbr,), device_id_type=pl.DeviceIdType.MESH,
            )
        pl.semaphore_wait(barrier_sem, 2)

        xv = x_ref[...]
        scores = jnp.dot(xv, rw_ref[...], preferred_element_type=jnp.float32)
        m = jnp.max(scores, axis=-1, keepdims=True)
        p = jnp.exp(scores - m)
        p = p / jnp.sum(p, axis=-1, keepdims=True)

        e0 = idx_ref[:, 0:1]
        e1 = idx_ref[:, 1:2]
        iota_e = lax.broadcasted_iota(jnp.int32, (T, E), 1)
        g0 = jnp.sum(jnp.where(iota_e == e0, p, 0.0), axis=-1, keepdims=True)
        g1 = jnp.sum(jnp.where(iota_e == e1, p, 0.0), axis=-1, keepdims=True)
        gs = g0 + g1
        w0 = g0 / gs
        w1 = g1 / gs

        acc = jnp.zeros((T, H), jnp.float32)
        for le in range(E_local):
            ge = my * E_local + le
            we = (w0 * (e0 == ge).astype(jnp.float32)
                  + w1 * (e1 == ge).astype(jnp.float32))
            acc = acc + jnp.dot(xv * we, ew_ref[le],
                                preferred_element_type=jnp.float32)
        partial_ref[...] = acc

        c0 = lax.rem(my - 1 + N_DEV, N_DEV)
        comm_ref[0] = partial_ref[pl.ds(c0 * rows_per, rows_per), :]
        for s in range(N_DEV - 1):
            send_slot = s % 2
            recv_slot = (s + 1) % 2
            rdma = pltpu.make_async_remote_copy(
                src_ref=comm_ref.at[send_slot],
                dst_ref=comm_ref.at[recv_slot],
                send_sem=send_sems.at[send_slot],
                recv_sem=recv_sems.at[recv_slot],
                device_id=(right,),
                device_id_type=pl.DeviceIdType.MESH,
            )
            rdma.start()
            rdma.wait()

            c = lax.rem(my - 2 - s + 2 * N_DEV, N_DEV)
            contrib = partial_ref[pl.ds(c * rows_per, rows_per), :]
            if s < N_DEV - 2:
                comm_ref[recv_slot] = comm_ref[recv_slot] + contrib
            else:
                out_ref[...] = comm_ref[recv_slot] + contrib

    return pl.pallas_call(
        body,
        out_shape=jax.ShapeDtypeStruct((rows_per, H), jnp.float32),
        in_specs=[
            pl.BlockSpec(memory_space=pltpu.VMEM),
            pl.BlockSpec(memory_space=pltpu.VMEM),
            pl.BlockSpec(memory_space=pltpu.VMEM),
            pl.BlockSpec(memory_space=pltpu.VMEM),
        ],
        out_specs=pl.BlockSpec(memory_space=pltpu.VMEM),
        scratch_shapes=[
            pltpu.VMEM((T, H), jnp.float32),
            pltpu.VMEM((2, rows_per, H), jnp.float32),
            pltpu.SemaphoreType.DMA((2,)),
            pltpu.SemaphoreType.DMA((2,)),
        ],
        compiler_params=pltpu.CompilerParams(collective_id=0),
    )(x, router_W, route_idx, expert_W)


# device time: 28795 ns/iter; 2.7246x vs baseline; 2.7246x over previous
import jax
import jax.numpy as jnp
from jax import lax
from jax.experimental import pallas as pl
from jax.experimental.pallas import tpu as pltpu

N_DEV = 32


def kernel(x, router_W, route_idx, expert_W):
    T, D = x.shape
    E = router_W.shape[1]
    E_local, _, H = expert_W.shape
    rows_per = T // N_DEV

    def body(x_ref, rw_ref, idx_ref, ew_ref, out_ref,
             partial_ref, gather_ref, send_sems, recv_sems):
        my = lax.axis_index("i")

        barrier_sem = pltpu.get_barrier_semaphore()
        for r in range(1, N_DEV):
            pl.semaphore_signal(
                barrier_sem, inc=1,
                device_id=(lax.rem(my + r, N_DEV),),
                device_id_type=pl.DeviceIdType.MESH,
            )
        pl.semaphore_wait(barrier_sem, N_DEV - 1)

        xv = x_ref[...]
        scores = jnp.dot(xv, rw_ref[...], preferred_element_type=jnp.float32)
        m = jnp.max(scores, axis=-1, keepdims=True)
        p = jnp.exp(scores - m)
        p = p / jnp.sum(p, axis=-1, keepdims=True)

        e0 = idx_ref[:, 0:1]
        e1 = idx_ref[:, 1:2]
        iota_e = lax.broadcasted_iota(jnp.int32, (T, E), 1)
        g0 = jnp.sum(jnp.where(iota_e == e0, p, 0.0), axis=-1, keepdims=True)
        g1 = jnp.sum(jnp.where(iota_e == e1, p, 0.0), axis=-1, keepdims=True)
        gs = g0 + g1
        w0 = g0 / gs
        w1 = g1 / gs

        acc = jnp.zeros((T, H), jnp.float32)
        for le in range(E_local):
            ge = my * E_local + le
            we = (w0 * (e0 == ge).astype(jnp.float32)
                  + w1 * (e1 == ge).astype(jnp.float32))
            acc = acc + jnp.dot(xv * we, ew_ref[le],
                                preferred_element_type=jnp.float32)
        partial_ref[...] = acc

        sends = []
        for r in range(1, N_DEV):
            dst = lax.rem(my + r, N_DEV)
            slot = N_DEV - r
            rdma = pltpu.make_async_remote_copy(
                src_ref=partial_ref.at[pl.ds(dst * rows_per, rows_per), :],
                dst_ref=gather_ref.at[slot],
                send_sem=send_sems.at[r],
                recv_sem=recv_sems.at[slot],
                device_id=(dst,),
                device_id_type=pl.DeviceIdType.MESH,
            )
            rdma.start()
            sends.append(rdma)

        gather_ref[pl.ds(0, 1)] = partial_ref[
            pl.ds(my * rows_per, rows_per), :].reshape(1, rows_per, H)

        for slot in range(1, N_DEV):
            recv = pltpu.make_async_remote_copy(
                src_ref=partial_ref.at[pl.ds(0, rows_per), :],
                dst_ref=gather_ref.at[slot],
                send_sem=send_sems.at[slot],
                recv_sem=recv_sems.at[slot],
                device_id=(my,),
                device_id_type=pl.DeviceIdType.MESH,
            )
            recv.wait_recv()

        out_ref[...] = jnp.sum(gather_ref[...], axis=0)

        for rdma in sends:
            rdma.wait_send()

    return pl.pallas_call(
        body,
        out_shape=jax.ShapeDtypeStruct((rows_per, H), jnp.float32),
        in_specs=[
            pl.BlockSpec(memory_space=pltpu.VMEM),
            pl.BlockSpec(memory_space=pltpu.VMEM),
            pl.BlockSpec(memory_space=pltpu.VMEM),
            pl.BlockSpec(memory_space=pltpu.VMEM),
        ],
        out_specs=pl.BlockSpec(memory_space=pltpu.VMEM),
        scratch_shapes=[
            pltpu.VMEM((T, H), jnp.float32),
            pltpu.VMEM((N_DEV, rows_per, H), jnp.float32),
            pltpu.SemaphoreType.DMA((N_DEV,)),
            pltpu.SemaphoreType.DMA((N_DEV,)),
        ],
        compiler_params=pltpu.CompilerParams(collective_id=0),
    )(x, router_W, route_idx, expert_W)
